# baseline (speedup 1.0000x reference)
import functools

import jax
import jax.numpy as jnp
from jax import lax
from jax.experimental import pallas as pl
from jax.experimental.pallas import tpu as pltpu
from jax.experimental.pallas import tpu_sc as plsc

_NUM_CLASSES = 100
_TOPK = 5
_BLK = 4096
_Q = 1024
_CHUNK = 128
_CPB = _BLK // _CHUNK
_GRP = 16
_IMAX = 2**31 - 1


def _chunkmin_kernel(n_train, xt_ref, xq_ref, out_ref):
    i = pl.program_id(0)
    blk = xt_ref.shape[0]
    q = xq_ref.shape[0]
    xq = xq_ref[...]
    xt = xt_ref[...]
    q_sq = jnp.sum(xq * xq, axis=1, keepdims=True)
    k_sq = jnp.sum(xt * xt, axis=1)[None, :]
    dot = lax.dot_general(xq, xt, (((1,), (1,)), ((), ())),
                          preferred_element_type=jnp.float32)
    d2 = q_sq + k_sq - 2.0 * dot
    gidx = i * blk + lax.broadcasted_iota(jnp.int32, (1, blk), 1)
    d2 = jnp.where(gidx < n_train, d2, jnp.inf)
    cmin2 = jnp.min(d2.reshape(q, _CPB, _CHUNK), axis=2)
    out_ref[...] = jnp.sqrt(jnp.maximum(cmin2, 0.0))[None]


def _chunksel_kernel(cmin_ref, out_ref):
    vals = cmin_ref[...]
    q, nc = vals.shape
    ciota = jnp.broadcast_to(
        lax.broadcasted_iota(jnp.int32, (1, nc), 1), (q, nc))
    picks = []
    for _ in range(_TOPK):
        m = jnp.min(vals, axis=1, keepdims=True)
        csel = jnp.min(jnp.where(vals == m, ciota, _IMAX),
                       axis=1, keepdims=True)
        vals = jnp.where(ciota == csel, jnp.inf, vals)
        picks.append(csel)
    out_ref[...] = jnp.concatenate(picks, axis=1)


def _make_gather(n_sel):
    info = plsc.get_sparse_core_info()
    nc, ns = info.num_cores, info.num_subcores
    nw = nc * ns
    per_w = n_sel // nw
    batch = 16
    n_batch = per_w // batch
    mesh = plsc.VectorSubcoreMesh(core_axis_name="c", subcore_axis_name="s")

    @functools.partial(
        pl.kernel, mesh=mesh,
        out_type=[
            jax.ShapeDtypeStruct((n_sel, _CHUNK * 32), jnp.float32),
            jax.ShapeDtypeStruct((n_sel, _CHUNK), jnp.int32),
        ],
        scratch_types=[
            pltpu.VMEM((batch,), jnp.int32),
            pltpu.VMEM((batch, _CHUNK * 32), jnp.float32),
            pltpu.VMEM((batch, _CHUNK), jnp.int32),
            pltpu.SemaphoreType.DMA,
        ],
    )
    def gather(xtab_hbm, ytab_hbm, idx_hbm, outx_hbm, outy_hbm,
               idx_v, xbuf, ybuf, sem):
        wid = lax.axis_index("s") * nc + lax.axis_index("c")
        base = wid * per_w
        for b in range(n_batch):
            off = base + b * batch
            pltpu.sync_copy(idx_hbm.at[pl.ds(off, batch)], idx_v)
            pltpu.async_copy(xtab_hbm.at[idx_v], xbuf, sem).wait()
            pltpu.sync_copy(xbuf, outx_hbm.at[pl.ds(off, batch)])
            pltpu.async_copy(ytab_hbm.at[idx_v], ybuf, sem).wait()
            pltpu.sync_copy(ybuf, outy_hbm.at[pl.ds(off, batch)])

    return gather


def _rerank_kernel(n_train, xq_ref, xg_ref, lab_ref, cid_ref, out_ref):
    g = xq_ref.shape[0]
    ncand = _TOPK * _CHUNK
    xq = xq_ref[...]
    xg = xg_ref[...]
    q_sq = jnp.sum(xq * xq, axis=1, keepdims=True)
    k_sq = jnp.sum(xg * xg, axis=1)[None, :]
    dot = lax.dot_general(xq, xg, (((1,), (1,)), ((), ())),
                          preferred_element_type=jnp.float32)
    d2 = q_sq + k_sq - 2.0 * dot
    dist_all = jnp.sqrt(jnp.maximum(d2, 0.0))
    d3 = dist_all.reshape(g, g, ncand)
    own = (lax.broadcasted_iota(jnp.int32, (g, g, 1), 0)
           == lax.broadcasted_iota(jnp.int32, (g, g, 1), 1))
    dist = jnp.min(jnp.where(own, d3, jnp.inf), axis=1)

    cid = cid_ref[0]
    lane = lax.broadcasted_iota(jnp.int32, (1, 1, _CHUNK), 2)
    idxs = (cid[:, :, None] * _CHUNK + lane).reshape(g, ncand)
    labs = lab_ref[...].reshape(g, ncand)
    dist = jnp.where(idxs < n_train, dist, jnp.inf)

    vals = dist
    sel_l = []
    for _ in range(_TOPK):
        m = jnp.min(vals, axis=1, keepdims=True)
        isel = jnp.min(jnp.where(vals == m, idxs, _IMAX),
                       axis=1, keepdims=True)
        sel = idxs == isel
        sel_l.append(jnp.min(jnp.where(sel, labs, _IMAX),
                             axis=1, keepdims=True))
        vals = jnp.where(sel, jnp.inf, vals)

    counts = []
    for a in range(_TOPK):
        c = jnp.zeros((g, 1), jnp.int32)
        for b in range(_TOPK):
            c = c + (sel_l[a] == sel_l[b]).astype(jnp.int32)
        counts.append(c)
    best_s = counts[0] * (_NUM_CLASSES * 10) - sel_l[0]
    best_l = sel_l[0]
    for a in range(1, _TOPK):
        s = counts[a] * (_NUM_CLASSES * 10) - sel_l[a]
        upd = s > best_s
        best_s = jnp.where(upd, s, best_s)
        best_l = jnp.where(upd, sel_l[a], best_l)
    out_ref[...] = best_l


@jax.jit
def kernel(X_train, X_test, y_train):
    n_train = X_train.shape[0]
    n_steps = (n_train + _BLK - 1) // _BLK
    n_pad = n_steps * _BLK
    n_chunks = n_pad // _CHUNK
    xt = jnp.pad(X_train, ((0, n_pad - n_train), (0, 0)))
    yt = jnp.pad(y_train, (0, n_pad - n_train))

    cmin3 = pl.pallas_call(
        functools.partial(_chunkmin_kernel, n_train),
        grid=(n_steps,),
        in_specs=[
            pl.BlockSpec((_BLK, 32), lambda i: (i, 0)),
            pl.BlockSpec((_Q, 32), lambda i: (0, 0)),
        ],
        out_specs=pl.BlockSpec((1, _Q, _CPB), lambda i: (i, 0, 0)),
        out_shape=jax.ShapeDtypeStruct((n_steps, _Q, _CPB), jnp.float32),
        compiler_params=pltpu.CompilerParams(
            dimension_semantics=("arbitrary",)),
    )(xt, X_test)
    cmin = cmin3.transpose(1, 0, 2).reshape(_Q, n_chunks)

    qtile = 256
    cids = pl.pallas_call(
        _chunksel_kernel,
        grid=(_Q // qtile,),
        in_specs=[pl.BlockSpec((qtile, n_chunks), lambda i: (i, 0))],
        out_specs=pl.BlockSpec((qtile, _TOPK), lambda i: (i, 0)),
        out_shape=jax.ShapeDtypeStruct((_Q, _TOPK), jnp.int32),
    )(cmin)

    n_sel = _Q * _TOPK
    idx_flat = cids.reshape(n_sel)
    xtab = xt.reshape(n_chunks, _CHUNK * 32)
    ytab = yt.reshape(n_chunks, _CHUNK)
    xg, yg = _make_gather(n_sel)(xtab, ytab, idx_flat)

    xg2 = xg.reshape(n_sel * _CHUNK, 32)
    cids3 = cids.reshape(_Q // _GRP, _GRP, _TOPK)
    out = pl.pallas_call(
        functools.partial(_rerank_kernel, n_train),
        grid=(_Q // _GRP,),
        in_specs=[
            pl.BlockSpec((_GRP, 32), lambda i: (i, 0)),
            pl.BlockSpec((_GRP * _TOPK * _CHUNK, 32), lambda i: (i, 0)),
            pl.BlockSpec((_GRP * _TOPK, _CHUNK), lambda i: (i, 0)),
            pl.BlockSpec((1, _GRP, _TOPK), lambda i: (i, 0, 0)),
        ],
        out_specs=pl.BlockSpec((_GRP, 1), lambda i: (i, 0)),
        out_shape=jax.ShapeDtypeStruct((_Q, 1), jnp.int32),
    )(X_test, xg2, yg, cids3)
    return out[:, 0]


# speedup vs baseline: 1.0539x; 1.0539x over previous
import functools

import jax
import jax.numpy as jnp
from jax import lax
from jax.experimental import pallas as pl
from jax.experimental.pallas import tpu as pltpu
from jax.experimental.pallas import tpu_sc as plsc

_NUM_CLASSES = 100
_TOPK = 5
_BLK = 4096
_Q = 1024
_CHUNK = 128
_CPB = _BLK // _CHUNK
_GRP = 32
_IMAX = 2**31 - 1


def _chunkmin_kernel(n_train, xt_ref, xq_ref, out_ref):
    i = pl.program_id(0)
    blk = xt_ref.shape[0]
    q = xq_ref.shape[0]
    xq = xq_ref[...]
    xt = xt_ref[...]
    q_sq = jnp.sum(xq * xq, axis=1, keepdims=True)
    k_sq = jnp.sum(xt * xt, axis=1)[None, :]
    dot = lax.dot_general(xq, xt, (((1,), (1,)), ((), ())),
                          preferred_element_type=jnp.float32)
    d2 = q_sq + k_sq - 2.0 * dot
    gidx = i * blk + lax.broadcasted_iota(jnp.int32, (1, blk), 1)
    d2 = jnp.where(gidx < n_train, d2, jnp.inf)
    cmin2 = jnp.min(d2.reshape(q, _CPB, _CHUNK), axis=2)
    out_ref[...] = jnp.sqrt(jnp.maximum(cmin2, 0.0))[None]


def _chunksel_kernel(cmin_ref, out_ref):
    vals = cmin_ref[...]
    q, nc = vals.shape
    ciota = jnp.broadcast_to(
        lax.broadcasted_iota(jnp.int32, (1, nc), 1), (q, nc))
    picks = []
    for _ in range(_TOPK):
        m = jnp.min(vals, axis=1, keepdims=True)
        csel = jnp.min(jnp.where(vals == m, ciota, _IMAX),
                       axis=1, keepdims=True)
        vals = jnp.where(ciota == csel, jnp.inf, vals)
        picks.append(csel)
    out_ref[...] = jnp.concatenate(picks, axis=1)


def _make_gather(n_sel):
    info = plsc.get_sparse_core_info()
    nc, ns = info.num_cores, info.num_subcores
    nw = nc * ns
    per_w = n_sel // nw
    batch = 16
    n_batch = per_w // batch
    mesh = plsc.VectorSubcoreMesh(core_axis_name="c", subcore_axis_name="s")

    @functools.partial(
        pl.kernel, mesh=mesh,
        out_type=[
            jax.ShapeDtypeStruct((n_sel, _CHUNK * 32), jnp.float32),
            jax.ShapeDtypeStruct((n_sel, _CHUNK), jnp.int32),
        ],
        scratch_types=[
            pltpu.VMEM((batch,), jnp.int32),
            pltpu.VMEM((batch, _CHUNK * 32), jnp.float32),
            pltpu.VMEM((batch, _CHUNK), jnp.int32),
            pltpu.SemaphoreType.DMA,
        ],
    )
    def gather(xtab_hbm, ytab_hbm, idx_hbm, outx_hbm, outy_hbm,
               idx_v, xbuf, ybuf, sem):
        wid = lax.axis_index("s") * nc + lax.axis_index("c")
        base = wid * per_w
        for b in range(n_batch):
            off = base + b * batch
            pltpu.sync_copy(idx_hbm.at[pl.ds(off, batch)], idx_v)
            pltpu.async_copy(xtab_hbm.at[idx_v], xbuf, sem).wait()
            pltpu.sync_copy(xbuf, outx_hbm.at[pl.ds(off, batch)])
            pltpu.async_copy(ytab_hbm.at[idx_v], ybuf, sem).wait()
            pltpu.sync_copy(ybuf, outy_hbm.at[pl.ds(off, batch)])

    return gather


def _rerank_kernel(n_train, xq_ref, xg_ref, lab_ref, cid_ref, out_ref):
    g = xq_ref.shape[0]
    ncand = _TOPK * _CHUNK
    xq = xq_ref[...]
    xg = xg_ref[...]
    q_sq = jnp.sum(xq * xq, axis=1, keepdims=True)
    k_sq = jnp.sum(xg * xg, axis=1)[None, :]
    dot = lax.dot_general(xq, xg, (((1,), (1,)), ((), ())),
                          preferred_element_type=jnp.float32)
    d2 = q_sq + k_sq - 2.0 * dot
    dist_all = jnp.sqrt(jnp.maximum(d2, 0.0))
    d3 = dist_all.reshape(g, g, ncand)
    own = (lax.broadcasted_iota(jnp.int32, (g, g, 1), 0)
           == lax.broadcasted_iota(jnp.int32, (g, g, 1), 1))
    dist = jnp.min(jnp.where(own, d3, jnp.inf), axis=1)

    cid = cid_ref[0]
    lane = lax.broadcasted_iota(jnp.int32, (1, 1, _CHUNK), 2)
    idxs = (cid[:, :, None] * _CHUNK + lane).reshape(g, ncand)
    labs = lab_ref[...].reshape(g, ncand)
    dist = jnp.where(idxs < n_train, dist, jnp.inf)

    vals = dist
    sel_l = []
    for _ in range(_TOPK):
        m = jnp.min(vals, axis=1, keepdims=True)
        isel = jnp.min(jnp.where(vals == m, idxs, _IMAX),
                       axis=1, keepdims=True)
        sel = idxs == isel
        sel_l.append(jnp.min(jnp.where(sel, labs, _IMAX),
                             axis=1, keepdims=True))
        vals = jnp.where(sel, jnp.inf, vals)

    counts = []
    for a in range(_TOPK):
        c = jnp.zeros((g, 1), jnp.int32)
        for b in range(_TOPK):
            c = c + (sel_l[a] == sel_l[b]).astype(jnp.int32)
        counts.append(c)
    best_s = counts[0] * (_NUM_CLASSES * 10) - sel_l[0]
    best_l = sel_l[0]
    for a in range(1, _TOPK):
        s = counts[a] * (_NUM_CLASSES * 10) - sel_l[a]
        upd = s > best_s
        best_s = jnp.where(upd, s, best_s)
        best_l = jnp.where(upd, sel_l[a], best_l)
    out_ref[...] = best_l


@jax.jit
def kernel(X_train, X_test, y_train):
    n_train = X_train.shape[0]
    n_steps = (n_train + _BLK - 1) // _BLK
    n_pad = n_steps * _BLK
    n_chunks = n_pad // _CHUNK
    xt = jnp.pad(X_train, ((0, n_pad - n_train), (0, 0)))
    yt = jnp.pad(y_train, (0, n_pad - n_train))

    cmin3 = pl.pallas_call(
        functools.partial(_chunkmin_kernel, n_train),
        grid=(n_steps,),
        in_specs=[
            pl.BlockSpec((_BLK, 32), lambda i: (i, 0)),
            pl.BlockSpec((_Q, 32), lambda i: (0, 0)),
        ],
        out_specs=pl.BlockSpec((1, _Q, _CPB), lambda i: (i, 0, 0)),
        out_shape=jax.ShapeDtypeStruct((n_steps, _Q, _CPB), jnp.float32),
        compiler_params=pltpu.CompilerParams(
            dimension_semantics=("arbitrary",)),
    )(xt, X_test)
    cmin = cmin3.transpose(1, 0, 2).reshape(_Q, n_chunks)

    qtile = 256
    cids = pl.pallas_call(
        _chunksel_kernel,
        grid=(_Q // qtile,),
        in_specs=[pl.BlockSpec((qtile, n_chunks), lambda i: (i, 0))],
        out_specs=pl.BlockSpec((qtile, _TOPK), lambda i: (i, 0)),
        out_shape=jax.ShapeDtypeStruct((_Q, _TOPK), jnp.int32),
    )(cmin)

    n_sel = _Q * _TOPK
    idx_flat = cids.reshape(n_sel)
    xtab = xt.reshape(n_chunks, _CHUNK * 32)
    ytab = yt.reshape(n_chunks, _CHUNK)
    xg, yg = _make_gather(n_sel)(xtab, ytab, idx_flat)

    xg2 = xg.reshape(n_sel * _CHUNK, 32)
    cids3 = cids.reshape(_Q // _GRP, _GRP, _TOPK)
    out = pl.pallas_call(
        functools.partial(_rerank_kernel, n_train),
        grid=(_Q // _GRP,),
        in_specs=[
            pl.BlockSpec((_GRP, 32), lambda i: (i, 0)),
            pl.BlockSpec((_GRP * _TOPK * _CHUNK, 32), lambda i: (i, 0)),
            pl.BlockSpec((_GRP * _TOPK, _CHUNK), lambda i: (i, 0)),
            pl.BlockSpec((1, _GRP, _TOPK), lambda i: (i, 0, 0)),
        ],
        out_specs=pl.BlockSpec((_GRP, 1), lambda i: (i, 0)),
        out_shape=jax.ShapeDtypeStruct((_Q, 1), jnp.int32),
    )(X_test, xg2, yg, cids3)
    return out[:, 0]


# speedup vs baseline: 1.0653x; 1.0109x over previous
import functools

import jax
import jax.numpy as jnp
from jax import lax
from jax.experimental import pallas as pl
from jax.experimental.pallas import tpu as pltpu
from jax.experimental.pallas import tpu_sc as plsc

_NUM_CLASSES = 100
_TOPK = 5
_BLK = 4096
_Q = 1024
_CHUNK = 128
_CPB = _BLK // _CHUNK
_GRP = 32
_IMAX = 2**31 - 1
_PADVAL = 1.0e18


def _chunkmin_kernel(xt_ref, xq_ref, out_ref):
    q = xq_ref.shape[0]
    xq = xq_ref[...]
    xt = xt_ref[...]
    q_sq = jnp.sum(xq * xq, axis=1, keepdims=True)
    k_sq = jnp.sum(xt * xt, axis=1)[None, :]
    dot = lax.dot_general(xq, xt, (((1,), (1,)), ((), ())),
                          preferred_element_type=jnp.float32)
    d2 = q_sq + k_sq - 2.0 * dot
    cmin2 = jnp.min(d2.reshape(q, _CPB, _CHUNK), axis=2)
    out_ref[...] = jnp.sqrt(jnp.maximum(cmin2, 0.0))[None]


def _chunksel_kernel(cmin_ref, out_ref):
    vals = cmin_ref[...]
    q, nc = vals.shape
    ciota = jnp.broadcast_to(
        lax.broadcasted_iota(jnp.int32, (1, nc), 1), (q, nc))
    picks = []
    for _ in range(_TOPK):
        m = jnp.min(vals, axis=1, keepdims=True)
        csel = jnp.min(jnp.where(vals == m, ciota, _IMAX),
                       axis=1, keepdims=True)
        vals = jnp.where(ciota == csel, jnp.inf, vals)
        picks.append(csel)
    out_ref[...] = jnp.concatenate(picks, axis=1)


def _make_gather(n_sel):
    info = plsc.get_sparse_core_info()
    nc, ns = info.num_cores, info.num_subcores
    nw = nc * ns
    per_w = n_sel // nw
    batch = 8
    n_batch = per_w // batch
    mesh = plsc.VectorSubcoreMesh(core_axis_name="c", subcore_axis_name="s")

    @functools.partial(
        pl.kernel, mesh=mesh,
        out_type=[
            jax.ShapeDtypeStruct((n_sel, _CHUNK * 32), jnp.float32),
            jax.ShapeDtypeStruct((n_sel, _CHUNK), jnp.int32),
        ],
        scratch_types=[
            pltpu.VMEM((batch,), jnp.int32),
            pltpu.VMEM((batch,), jnp.int32),
            pltpu.VMEM((batch, _CHUNK * 32), jnp.float32),
            pltpu.VMEM((batch, _CHUNK * 32), jnp.float32),
            pltpu.VMEM((batch, _CHUNK), jnp.int32),
            pltpu.VMEM((batch, _CHUNK), jnp.int32),
            pltpu.SemaphoreType.DMA,
            pltpu.SemaphoreType.DMA,
        ],
    )
    def gather(xtab_hbm, ytab_hbm, idx_hbm, outx_hbm, outy_hbm,
               idx_a, idx_b, xbuf_a, xbuf_b, ybuf_a, ybuf_b, sg, sw):
        wid = lax.axis_index("s") * nc + lax.axis_index("c")
        base = wid * per_w
        bufs = ((idx_a, xbuf_a, ybuf_a), (idx_b, xbuf_b, ybuf_b))
        pending = [None, None]
        for b in range(n_batch):
            k = b % 2
            idx_v, xbuf, ybuf = bufs[k]
            if pending[k] is not None:
                pending[k][0].wait()
                pending[k][1].wait()
            off = base + b * batch
            pltpu.sync_copy(idx_hbm.at[pl.ds(off, batch)], idx_v)
            ax = pltpu.async_copy(xtab_hbm.at[idx_v], xbuf, sg)
            ay = pltpu.async_copy(ytab_hbm.at[idx_v], ybuf, sg)
            ax.wait()
            ay.wait()
            wx = pltpu.async_copy(xbuf, outx_hbm.at[pl.ds(off, batch)], sw)
            wy = pltpu.async_copy(ybuf, outy_hbm.at[pl.ds(off, batch)], sw)
            pending[k] = (wx, wy)
        for p in pending:
            if p is not None:
                p[0].wait()
                p[1].wait()

    return gather


def _rerank_kernel(n_train, xq_ref, xg_ref, lab_ref, cid_ref, out_ref):
    g = xq_ref.shape[0]
    ncand = _TOPK * _CHUNK
    xq = xq_ref[...]
    xg = xg_ref[...]
    q_sq = jnp.sum(xq * xq, axis=1, keepdims=True)
    dot = lax.dot_general(xq, xg, (((1,), (1,)), ((), ())),
                          preferred_element_type=jnp.float32)
    dot3 = dot.reshape(g, g, ncand)
    own = (lax.broadcasted_iota(jnp.int32, (g, g, 1), 0)
           == lax.broadcasted_iota(jnp.int32, (g, g, 1), 1))
    dot_own = jnp.sum(jnp.where(own, dot3, 0.0), axis=1)
    k_sq_own = jnp.sum(xg * xg, axis=1).reshape(g, ncand)
    d2 = q_sq + k_sq_own - 2.0 * dot_own
    dist = jnp.sqrt(jnp.maximum(d2, 0.0))

    cid = cid_ref[0]
    lane = lax.broadcasted_iota(jnp.int32, (1, 1, _CHUNK), 2)
    idxs = (cid[:, :, None] * _CHUNK + lane).reshape(g, ncand)
    labs = lab_ref[...].reshape(g, ncand)
    dist = jnp.where(idxs < n_train, dist, jnp.inf)

    vals = dist
    sel_l = []
    for _ in range(_TOPK):
        m = jnp.min(vals, axis=1, keepdims=True)
        isel = jnp.min(jnp.where(vals == m, idxs, _IMAX),
                       axis=1, keepdims=True)
        sel = idxs == isel
        sel_l.append(jnp.min(jnp.where(sel, labs, _IMAX),
                             axis=1, keepdims=True))
        vals = jnp.where(sel, jnp.inf, vals)

    counts = []
    for a in range(_TOPK):
        c = jnp.zeros((g, 1), jnp.int32)
        for b in range(_TOPK):
            c = c + (sel_l[a] == sel_l[b]).astype(jnp.int32)
        counts.append(c)
    best_s = counts[0] * (_NUM_CLASSES * 10) - sel_l[0]
    best_l = sel_l[0]
    for a in range(1, _TOPK):
        s = counts[a] * (_NUM_CLASSES * 10) - sel_l[a]
        upd = s > best_s
        best_s = jnp.where(upd, s, best_s)
        best_l = jnp.where(upd, sel_l[a], best_l)
    out_ref[...] = best_l


@jax.jit
def kernel(X_train, X_test, y_train):
    n_train = X_train.shape[0]
    n_steps = (n_train + _BLK - 1) // _BLK
    n_pad = n_steps * _BLK
    n_chunks = n_pad // _CHUNK
    xt = jnp.pad(X_train, ((0, n_pad - n_train), (0, 0)),
                 constant_values=_PADVAL)
    yt = jnp.pad(y_train, (0, n_pad - n_train))

    cmin3 = pl.pallas_call(
        _chunkmin_kernel,
        grid=(n_steps,),
        in_specs=[
            pl.BlockSpec((_BLK, 32), lambda i: (i, 0)),
            pl.BlockSpec((_Q, 32), lambda i: (0, 0)),
        ],
        out_specs=pl.BlockSpec((1, _Q, _CPB), lambda i: (i, 0, 0)),
        out_shape=jax.ShapeDtypeStruct((n_steps, _Q, _CPB), jnp.float32),
        compiler_params=pltpu.CompilerParams(
            dimension_semantics=("arbitrary",)),
    )(xt, X_test)
    cmin = cmin3.transpose(1, 0, 2).reshape(_Q, n_chunks)

    qtile = 256
    cids = pl.pallas_call(
        _chunksel_kernel,
        grid=(_Q // qtile,),
        in_specs=[pl.BlockSpec((qtile, n_chunks), lambda i: (i, 0))],
        out_specs=pl.BlockSpec((qtile, _TOPK), lambda i: (i, 0)),
        out_shape=jax.ShapeDtypeStruct((_Q, _TOPK), jnp.int32),
    )(cmin)

    n_sel = _Q * _TOPK
    idx_flat = cids.reshape(n_sel)
    xtab = xt.reshape(n_chunks, _CHUNK * 32)
    ytab = yt.reshape(n_chunks, _CHUNK)
    xg, yg = _make_gather(n_sel)(xtab, ytab, idx_flat)

    xg2 = xg.reshape(n_sel * _CHUNK, 32)
    cids3 = cids.reshape(_Q // _GRP, _GRP, _TOPK)
    out = pl.pallas_call(
        functools.partial(_rerank_kernel, n_train),
        grid=(_Q // _GRP,),
        in_specs=[
            pl.BlockSpec((_GRP, 32), lambda i: (i, 0)),
            pl.BlockSpec((_GRP * _TOPK * _CHUNK, 32), lambda i: (i, 0)),
            pl.BlockSpec((_GRP * _TOPK, _CHUNK), lambda i: (i, 0)),
            pl.BlockSpec((1, _GRP, _TOPK), lambda i: (i, 0, 0)),
        ],
        out_specs=pl.BlockSpec((_GRP, 1), lambda i: (i, 0)),
        out_shape=jax.ShapeDtypeStruct((_Q, 1), jnp.int32),
    )(X_test, xg2, yg, cids3)
    return out[:, 0]


# speedup vs baseline: 1.0887x; 1.0219x over previous
import functools

import jax
import jax.numpy as jnp
from jax import lax
from jax.experimental import pallas as pl
from jax.experimental.pallas import tpu as pltpu
from jax.experimental.pallas import tpu_sc as plsc

_NUM_CLASSES = 100
_TOPK = 5
_BLK = 4096
_Q = 1024
_CHUNK = 128
_CPB = _BLK // _CHUNK
_GRP = 64
_IMAX = 2**31 - 1
_PADVAL = 1.0e18


def _chunkmin_kernel(xt_ref, xq_ref, out_ref):
    q = xq_ref.shape[0]
    xq = xq_ref[...]
    xt = xt_ref[...]
    q_sq = jnp.sum(xq * xq, axis=1, keepdims=True)
    k_sq = jnp.sum(xt * xt, axis=1)[None, :]
    dot = lax.dot_general(xq, xt, (((1,), (1,)), ((), ())),
                          preferred_element_type=jnp.float32)
    d2 = q_sq + k_sq - 2.0 * dot
    cmin2 = jnp.min(d2.reshape(q, _CPB, _CHUNK), axis=2)
    out_ref[...] = jnp.sqrt(jnp.maximum(cmin2, 0.0))[None]


def _chunksel_kernel(cmin_ref, out_ref):
    vals = cmin_ref[...]
    q, nc = vals.shape
    ciota = jnp.broadcast_to(
        lax.broadcasted_iota(jnp.int32, (1, nc), 1), (q, nc))
    picks = []
    for _ in range(_TOPK):
        m = jnp.min(vals, axis=1, keepdims=True)
        csel = jnp.min(jnp.where(vals == m, ciota, _IMAX),
                       axis=1, keepdims=True)
        vals = jnp.where(ciota == csel, jnp.inf, vals)
        picks.append(csel)
    out_ref[...] = jnp.concatenate(picks, axis=1)


def _make_gather(n_sel):
    info = plsc.get_sparse_core_info()
    nc, ns = info.num_cores, info.num_subcores
    nw = nc * ns
    per_w = n_sel // nw
    batch = 8
    n_batch = per_w // batch
    mesh = plsc.VectorSubcoreMesh(core_axis_name="c", subcore_axis_name="s")

    @functools.partial(
        pl.kernel, mesh=mesh,
        out_type=[
            jax.ShapeDtypeStruct((n_sel, _CHUNK * 32), jnp.float32),
            jax.ShapeDtypeStruct((n_sel, _CHUNK), jnp.int32),
        ],
        scratch_types=[
            pltpu.VMEM((batch,), jnp.int32),
            pltpu.VMEM((batch,), jnp.int32),
            pltpu.VMEM((batch, _CHUNK * 32), jnp.float32),
            pltpu.VMEM((batch, _CHUNK * 32), jnp.float32),
            pltpu.VMEM((batch, _CHUNK), jnp.int32),
            pltpu.VMEM((batch, _CHUNK), jnp.int32),
            pltpu.SemaphoreType.DMA,
            pltpu.SemaphoreType.DMA,
        ],
    )
    def gather(xtab_hbm, ytab_hbm, idx_hbm, outx_hbm, outy_hbm,
               idx_a, idx_b, xbuf_a, xbuf_b, ybuf_a, ybuf_b, sg, sw):
        wid = lax.axis_index("s") * nc + lax.axis_index("c")
        base = wid * per_w
        bufs = ((idx_a, xbuf_a, ybuf_a), (idx_b, xbuf_b, ybuf_b))
        pending = [None, None]
        for b in range(n_batch):
            k = b % 2
            idx_v, xbuf, ybuf = bufs[k]
            if pending[k] is not None:
                pending[k][0].wait()
                pending[k][1].wait()
            off = base + b * batch
            pltpu.sync_copy(idx_hbm.at[pl.ds(off, batch)], idx_v)
            ax = pltpu.async_copy(xtab_hbm.at[idx_v], xbuf, sg)
            ay = pltpu.async_copy(ytab_hbm.at[idx_v], ybuf, sg)
            ax.wait()
            ay.wait()
            wx = pltpu.async_copy(xbuf, outx_hbm.at[pl.ds(off, batch)], sw)
            wy = pltpu.async_copy(ybuf, outy_hbm.at[pl.ds(off, batch)], sw)
            pending[k] = (wx, wy)
        for p in pending:
            if p is not None:
                p[0].wait()
                p[1].wait()

    return gather


def _rerank_kernel(n_train, xq_ref, xg_ref, lab_ref, cid_ref, out_ref):
    g = xq_ref.shape[0]
    ncand = _TOPK * _CHUNK
    xq = xq_ref[...]
    xg = xg_ref[...]
    q_sq = jnp.sum(xq * xq, axis=1, keepdims=True)
    dot = lax.dot_general(xq, xg, (((1,), (1,)), ((), ())),
                          preferred_element_type=jnp.float32)
    dot3 = dot.reshape(g, g, ncand)
    own = (lax.broadcasted_iota(jnp.int32, (g, g, 1), 0)
           == lax.broadcasted_iota(jnp.int32, (g, g, 1), 1))
    dot_own = jnp.sum(jnp.where(own, dot3, 0.0), axis=1)
    k_sq_own = jnp.sum(xg * xg, axis=1).reshape(g, ncand)
    d2 = q_sq + k_sq_own - 2.0 * dot_own
    dist = jnp.sqrt(jnp.maximum(d2, 0.0))

    cid = cid_ref[0]
    lane = lax.broadcasted_iota(jnp.int32, (1, 1, _CHUNK), 2)
    idxs = (cid[:, :, None] * _CHUNK + lane).reshape(g, ncand)
    labs = lab_ref[...].reshape(g, ncand)
    dist = jnp.where(idxs < n_train, dist, jnp.inf)

    vals = dist
    sel_l = []
    for _ in range(_TOPK):
        m = jnp.min(vals, axis=1, keepdims=True)
        isel = jnp.min(jnp.where(vals == m, idxs, _IMAX),
                       axis=1, keepdims=True)
        sel = idxs == isel
        sel_l.append(jnp.min(jnp.where(sel, labs, _IMAX),
                             axis=1, keepdims=True))
        vals = jnp.where(sel, jnp.inf, vals)

    counts = []
    for a in range(_TOPK):
        c = jnp.zeros((g, 1), jnp.int32)
        for b in range(_TOPK):
            c = c + (sel_l[a] == sel_l[b]).astype(jnp.int32)
        counts.append(c)
    best_s = counts[0] * (_NUM_CLASSES * 10) - sel_l[0]
    best_l = sel_l[0]
    for a in range(1, _TOPK):
        s = counts[a] * (_NUM_CLASSES * 10) - sel_l[a]
        upd = s > best_s
        best_s = jnp.where(upd, s, best_s)
        best_l = jnp.where(upd, sel_l[a], best_l)
    out_ref[...] = best_l


@jax.jit
def kernel(X_train, X_test, y_train):
    n_train = X_train.shape[0]
    n_steps = (n_train + _BLK - 1) // _BLK
    n_pad = n_steps * _BLK
    n_chunks = n_pad // _CHUNK
    xt = jnp.pad(X_train, ((0, n_pad - n_train), (0, 0)),
                 constant_values=_PADVAL)
    yt = jnp.pad(y_train, (0, n_pad - n_train))

    cmin3 = pl.pallas_call(
        _chunkmin_kernel,
        grid=(n_steps,),
        in_specs=[
            pl.BlockSpec((_BLK, 32), lambda i: (i, 0)),
            pl.BlockSpec((_Q, 32), lambda i: (0, 0)),
        ],
        out_specs=pl.BlockSpec((1, _Q, _CPB), lambda i: (i, 0, 0)),
        out_shape=jax.ShapeDtypeStruct((n_steps, _Q, _CPB), jnp.float32),
        compiler_params=pltpu.CompilerParams(
            dimension_semantics=("parallel",)),
    )(xt, X_test)
    cmin = cmin3.transpose(1, 0, 2).reshape(_Q, n_chunks)

    qtile = 256
    cids = pl.pallas_call(
        _chunksel_kernel,
        grid=(_Q // qtile,),
        in_specs=[pl.BlockSpec((qtile, n_chunks), lambda i: (i, 0))],
        out_specs=pl.BlockSpec((qtile, _TOPK), lambda i: (i, 0)),
        out_shape=jax.ShapeDtypeStruct((_Q, _TOPK), jnp.int32),
    )(cmin)

    n_sel = _Q * _TOPK
    idx_flat = cids.reshape(n_sel)
    xtab = xt.reshape(n_chunks, _CHUNK * 32)
    ytab = yt.reshape(n_chunks, _CHUNK)
    xg, yg = _make_gather(n_sel)(xtab, ytab, idx_flat)

    xg2 = xg.reshape(n_sel * _CHUNK, 32)
    cids3 = cids.reshape(_Q // _GRP, _GRP, _TOPK)
    out = pl.pallas_call(
        functools.partial(_rerank_kernel, n_train),
        grid=(_Q // _GRP,),
        in_specs=[
            pl.BlockSpec((_GRP, 32), lambda i: (i, 0)),
            pl.BlockSpec((_GRP * _TOPK * _CHUNK, 32), lambda i: (i, 0)),
            pl.BlockSpec((_GRP * _TOPK, _CHUNK), lambda i: (i, 0)),
            pl.BlockSpec((1, _GRP, _TOPK), lambda i: (i, 0, 0)),
        ],
        out_specs=pl.BlockSpec((_GRP, 1), lambda i: (i, 0)),
        out_shape=jax.ShapeDtypeStruct((_Q, 1), jnp.int32),
        compiler_params=pltpu.CompilerParams(
            dimension_semantics=("parallel",)),
    )(X_test, xg2, yg, cids3)
    return out[:, 0]


# speedup vs baseline: 1.3717x; 1.2600x over previous
import functools

import jax
import jax.numpy as jnp
from jax import lax
from jax.experimental import pallas as pl
from jax.experimental.pallas import tpu as pltpu
from jax.experimental.pallas import tpu_sc as plsc

_NUM_CLASSES = 100
_TOPK = 5
_BLK = 4096
_Q = 1024
_CHUNK = 128
_CPB = _BLK // _CHUNK
_GRP = 64
_IMAX = 2**31 - 1
_PADVAL = 1.0e18


def _chunkmin_kernel(xt_ref, xq_ref, out_ref, ksq_ref):
    q = xq_ref.shape[0]
    xq = xq_ref[...]
    xt = xt_ref[...]
    q_sq = jnp.sum(xq * xq, axis=1, keepdims=True)
    k_sq = jnp.sum(xt * xt, axis=1)[None, :]
    dot = lax.dot_general(xq, xt, (((1,), (1,)), ((), ())),
                          preferred_element_type=jnp.float32)
    d2 = q_sq + k_sq - 2.0 * dot
    cmin2 = jnp.min(d2.reshape(q, _CPB, _CHUNK), axis=2)
    out_ref[...] = jnp.sqrt(jnp.maximum(cmin2, 0.0))[None]
    ksq_ref[...] = k_sq[None]


def _chunksel_kernel(cmin_ref, out_ref):
    vals = cmin_ref[...]
    q, nc = vals.shape
    ciota = jnp.broadcast_to(
        lax.broadcasted_iota(jnp.int32, (1, nc), 1), (q, nc))
    picks = []
    for _ in range(_TOPK):
        m = jnp.min(vals, axis=1, keepdims=True)
        csel = jnp.min(jnp.where(vals == m, ciota, _IMAX),
                       axis=1, keepdims=True)
        vals = jnp.where(ciota == csel, jnp.inf, vals)
        picks.append(csel)
    out_ref[...] = jnp.concatenate(picks, axis=1)


def _make_gather(n_sel):
    info = plsc.get_sparse_core_info()
    nc, ns = info.num_cores, info.num_subcores
    nw = nc * ns
    per_w = n_sel // nw
    batch = 8
    n_batch = per_w // batch
    mesh = plsc.VectorSubcoreMesh(core_axis_name="c", subcore_axis_name="s")

    @functools.partial(
        pl.kernel, mesh=mesh,
        out_type=[
            jax.ShapeDtypeStruct((n_sel, _CHUNK * 32), jnp.float32),
            jax.ShapeDtypeStruct((n_sel, _CHUNK), jnp.int32),
            jax.ShapeDtypeStruct((n_sel, _CHUNK), jnp.float32),
        ],
        scratch_types=[
            pltpu.VMEM((batch,), jnp.int32),
            pltpu.VMEM((batch,), jnp.int32),
            pltpu.VMEM((batch, _CHUNK * 32), jnp.float32),
            pltpu.VMEM((batch, _CHUNK * 32), jnp.float32),
            pltpu.VMEM((batch, _CHUNK), jnp.int32),
            pltpu.VMEM((batch, _CHUNK), jnp.int32),
            pltpu.VMEM((batch, _CHUNK), jnp.float32),
            pltpu.VMEM((batch, _CHUNK), jnp.float32),
            pltpu.SemaphoreType.DMA,
            pltpu.SemaphoreType.DMA,
        ],
    )
    def gather(xtab_hbm, ytab_hbm, ktab_hbm, idx_hbm,
               outx_hbm, outy_hbm, outk_hbm,
               idx_a, idx_b, xbuf_a, xbuf_b, ybuf_a, ybuf_b,
               kbuf_a, kbuf_b, sg, sw):
        wid = lax.axis_index("s") * nc + lax.axis_index("c")
        base = wid * per_w
        bufs = ((idx_a, xbuf_a, ybuf_a, kbuf_a),
                (idx_b, xbuf_b, ybuf_b, kbuf_b))
        pending = [None, None]
        for b in range(n_batch):
            k = b % 2
            idx_v, xbuf, ybuf, kbuf = bufs[k]
            if pending[k] is not None:
                for w in pending[k]:
                    w.wait()
            off = base + b * batch
            pltpu.sync_copy(idx_hbm.at[pl.ds(off, batch)], idx_v)
            ax = pltpu.async_copy(xtab_hbm.at[idx_v], xbuf, sg)
            ay = pltpu.async_copy(ytab_hbm.at[idx_v], ybuf, sg)
            ak = pltpu.async_copy(ktab_hbm.at[idx_v], kbuf, sg)
            ax.wait()
            ay.wait()
            ak.wait()
            pending[k] = (
                pltpu.async_copy(xbuf, outx_hbm.at[pl.ds(off, batch)], sw),
                pltpu.async_copy(ybuf, outy_hbm.at[pl.ds(off, batch)], sw),
                pltpu.async_copy(kbuf, outk_hbm.at[pl.ds(off, batch)], sw),
            )
        for p in pending:
            if p is not None:
                for w in p:
                    w.wait()

    return gather


def _rerank_kernel(n_train, xq_ref, xg_ref, lab_ref, ksq_ref, cid_ref,
                   out_ref):
    g = xq_ref.shape[0]
    ncand = _TOPK * _CHUNK
    xq = xq_ref[...]
    xg = xg_ref[...]
    q_sq = jnp.sum(xq * xq, axis=1, keepdims=True)
    dot = lax.dot_general(xq, xg, (((1,), (1,)), ((), ())),
                          preferred_element_type=jnp.float32)
    dot4 = dot.reshape(g, g, _TOPK, _CHUNK)
    own = (lax.broadcasted_iota(jnp.int32, (g, g, 1, 1), 0)
           == lax.broadcasted_iota(jnp.int32, (g, g, 1, 1), 1))
    dot_own = jnp.sum(jnp.where(own, dot4, 0.0),
                      axis=1).reshape(g, ncand)
    k_sq_own = ksq_ref[...].reshape(g, ncand)
    d2 = q_sq + k_sq_own - 2.0 * dot_own
    dist = jnp.sqrt(jnp.maximum(d2, 0.0))

    cid = cid_ref[0]
    lane = lax.broadcasted_iota(jnp.int32, (1, 1, _CHUNK), 2)
    idxs = (cid[:, :, None] * _CHUNK + lane).reshape(g, ncand)
    labs = lab_ref[...].reshape(g, ncand)
    dist = jnp.where(idxs < n_train, dist, jnp.inf)

    vals = dist
    sel_l = []
    for _ in range(_TOPK):
        m = jnp.min(vals, axis=1, keepdims=True)
        isel = jnp.min(jnp.where(vals == m, idxs, _IMAX),
                       axis=1, keepdims=True)
        sel = idxs == isel
        sel_l.append(jnp.min(jnp.where(sel, labs, _IMAX),
                             axis=1, keepdims=True))
        vals = jnp.where(sel, jnp.inf, vals)

    counts = []
    for a in range(_TOPK):
        c = jnp.zeros((g, 1), jnp.int32)
        for b in range(_TOPK):
            c = c + (sel_l[a] == sel_l[b]).astype(jnp.int32)
        counts.append(c)
    best_s = counts[0] * (_NUM_CLASSES * 10) - sel_l[0]
    best_l = sel_l[0]
    for a in range(1, _TOPK):
        s = counts[a] * (_NUM_CLASSES * 10) - sel_l[a]
        upd = s > best_s
        best_s = jnp.where(upd, s, best_s)
        best_l = jnp.where(upd, sel_l[a], best_l)
    out_ref[...] = best_l


@jax.jit
def kernel(X_train, X_test, y_train):
    n_train = X_train.shape[0]
    n_steps = (n_train + _BLK - 1) // _BLK
    n_pad = n_steps * _BLK
    n_chunks = n_pad // _CHUNK
    xt = jnp.pad(X_train, ((0, n_pad - n_train), (0, 0)),
                 constant_values=_PADVAL)
    yt = jnp.pad(y_train, (0, n_pad - n_train))

    cmin3 = pl.pallas_call(
        _chunkmin_kernel,
        grid=(n_steps,),
        in_specs=[
            pl.BlockSpec((_BLK, 32), lambda i: (i, 0)),
            pl.BlockSpec((_Q, 32), lambda i: (0, 0)),
        ],
        out_specs=[
            pl.BlockSpec((1, _Q, _CPB), lambda i: (i, 0, 0)),
            pl.BlockSpec((1, 1, _BLK), lambda i: (i, 0, 0)),
        ],
        out_shape=[
            jax.ShapeDtypeStruct((n_steps, _Q, _CPB), jnp.float32),
            jax.ShapeDtypeStruct((n_steps, 1, _BLK), jnp.float32),
        ],
        compiler_params=pltpu.CompilerParams(
            dimension_semantics=("parallel",)),
    )(xt, X_test)
    cmin3, ksq3 = cmin3
    cmin = cmin3.transpose(1, 0, 2).reshape(_Q, n_chunks)

    qtile = 256
    cids = pl.pallas_call(
        _chunksel_kernel,
        grid=(_Q // qtile,),
        in_specs=[pl.BlockSpec((qtile, n_chunks), lambda i: (i, 0))],
        out_specs=pl.BlockSpec((qtile, _TOPK), lambda i: (i, 0)),
        out_shape=jax.ShapeDtypeStruct((_Q, _TOPK), jnp.int32),
    )(cmin)

    n_sel = _Q * _TOPK
    idx_flat = cids.reshape(n_sel)
    xtab = xt.reshape(n_chunks, _CHUNK, 32).transpose(0, 2, 1)\
             .reshape(n_chunks, _CHUNK * 32)
    ytab = yt.reshape(n_chunks, _CHUNK)
    ktab = ksq3.reshape(n_chunks, _CHUNK)
    xg, yg, kg = _make_gather(n_sel)(xtab, ytab, ktab, idx_flat)

    xg3 = xg.reshape(n_sel, 32, _CHUNK)
    cids3 = cids.reshape(_Q // _GRP, _GRP, _TOPK)
    out = pl.pallas_call(
        functools.partial(_rerank_kernel, n_train),
        grid=(_Q // _GRP,),
        in_specs=[
            pl.BlockSpec((_GRP, 32), lambda i: (i, 0)),
            pl.BlockSpec((_GRP * _TOPK, 32, _CHUNK), lambda i: (i, 0, 0)),
            pl.BlockSpec((_GRP * _TOPK, _CHUNK), lambda i: (i, 0)),
            pl.BlockSpec((_GRP * _TOPK, _CHUNK), lambda i: (i, 0)),
            pl.BlockSpec((1, _GRP, _TOPK), lambda i: (i, 0, 0)),
        ],
        out_specs=pl.BlockSpec((_GRP, 1), lambda i: (i, 0)),
        out_shape=jax.ShapeDtypeStruct((_Q, 1), jnp.int32),
        compiler_params=pltpu.CompilerParams(
            dimension_semantics=("parallel",)),
    )(X_test, xg3, yg, kg, cids3)
    return out[:, 0]
